# baseline (speedup 1.0000x reference)
import numpy as np

import jax
import jax.numpy as jnp
from jax.experimental import pallas as pl
from jax.experimental.pallas import tpu as pltpu

_ALPHA = 1.0
_TOPK = 3
_BLOCK = 1024
_NEG = -1e9
_L = 128


def _perm_mats():
    p = np.zeros((_TOPK, _TOPK * _L, _L), np.float32)
    for w in range(_TOPK):
        for c in range(_L):
            q = _L * w + c
            p[w, _L * (q % _TOPK) + q // _TOPK, c] = 1.0
    return jnp.asarray(p)


def _hyper_kernel(mask_ref, ne_ref, hy_ref, p_ref, ew_ref, ei_ref):
    i = pl.program_id(0)
    b = ne_ref.shape[0]
    rows_out = ew_ref.shape[0]
    ne = ne_ref[...]
    hy = hy_ref[...]
    simt = jax.lax.dot_general(
        hy, ne, (((1,), (1,)), ((), ())),
        preferred_element_type=jnp.float32)
    simt = jnp.maximum(_ALPHA * simt, 0.0)
    avg = jnp.mean(mask_ref[...], axis=0)
    simt = jnp.where(avg[None, :] < 0.5, _NEG, simt)
    m = jnp.max(simt, axis=0, keepdims=True)
    e = jnp.exp(simt - m)
    soft = e / jnp.sum(e, axis=0, keepdims=True)

    h = soft.shape[0]
    row = jax.lax.broadcasted_iota(jnp.int32, soft.shape, 0)
    v = soft
    vks, iks = [], []
    for k in range(_TOPK):
        mk = jnp.max(v, axis=0)
        ik = jnp.min(jnp.where(v == mk[None, :], row, h), axis=0)
        vks.append(mk)
        iks.append(ik)
        v = jnp.where(row == ik[None, :], -1.0, v)

    vf = jnp.concatenate([x.reshape(b // _L, _L) for x in vks], axis=1)
    vi = jnp.concatenate([x.astype(jnp.float32).reshape(b // _L, _L)
                          for x in iks], axis=1)
    for w in range(_TOPK):
        pw = p_ref[w]
        tf = jax.lax.dot_general(vf, pw, (((1,), (0,)), ((), ())),
                                 precision=jax.lax.Precision.HIGHEST,
                                 preferred_element_type=jnp.float32)
        ti = jax.lax.dot_general(vi, pw, (((1,), (0,)), ((), ())),
                                 precision=jax.lax.Precision.HIGHEST,
                                 preferred_element_type=jnp.float32)
        ew_ref[pl.Slice(w, b // _L, _TOPK), :] = tf
        ei_ref[pl.ds(1, 1), pl.Slice(w, b // _L, _TOPK), :] = (
            ti.astype(jnp.int32)[None])

    ploc = (jax.lax.broadcasted_iota(jnp.int32, (rows_out, _L), 0) * _L
            + jax.lax.broadcasted_iota(jnp.int32, (rows_out, _L), 1))
    ei_ref[pl.ds(0, 1), :, :] = (b * i + ((ploc * 21846) >> 16))[None]


def kernel(features, mask, node_embeds, hyper_embeds):
    seq_len = min(features.shape[1], node_embeds.shape[0])
    ne = node_embeds[:seq_len]
    dim = ne.shape[1]
    hnum = hyper_embeds.shape[0]
    nblk = seq_len // _BLOCK
    rpb = _TOPK * _BLOCK // _L
    nrows = nblk * rpb

    ew, ei = pl.pallas_call(
        _hyper_kernel,
        grid=(nblk,),
        in_specs=[
            pl.BlockSpec((mask.shape[0], _BLOCK), lambda i: (0, i)),
            pl.BlockSpec((_BLOCK, dim), lambda i: (i, 0)),
            pl.BlockSpec((hnum, dim), lambda i: (0, 0)),
            pl.BlockSpec((_TOPK, _TOPK * _L, _L), lambda i: (0, 0, 0)),
        ],
        out_specs=[
            pl.BlockSpec((rpb, _L), lambda i: (i, 0)),
            pl.BlockSpec((2, rpb, _L), lambda i: (0, i, 0)),
        ],
        out_shape=[
            jax.ShapeDtypeStruct((nrows, _L), jnp.float32),
            jax.ShapeDtypeStruct((2, nrows, _L), jnp.int32),
        ],
    )(mask, ne, hyper_embeds, _perm_mats())

    return (ei.reshape(2, -1), ew.reshape(-1))


# speedup vs baseline: 1.3666x; 1.3666x over previous
import numpy as np

import jax
import jax.numpy as jnp
from jax.experimental import pallas as pl
from jax.experimental.pallas import tpu as pltpu

_ALPHA = 1.0
_TOPK = 3
_BLOCK = 2048
_NEG = -1e9
_L = 128


def _perm_mats():
    p = np.zeros((_TOPK, _TOPK * _L, _L), np.float32)
    for w in range(_TOPK):
        for c in range(_L):
            q = _L * w + c
            p[w, _L * (q % _TOPK) + q // _TOPK, c] = 1.0
    return jnp.asarray(p)


def _hyper_kernel(mask_ref, ne_ref, hy_ref, p_ref, ew_ref, ei_ref):
    i = pl.program_id(0)
    b = ne_ref.shape[0]
    rows_out = ew_ref.shape[0]
    ne = ne_ref[...]
    hy = hy_ref[...]
    simt = jax.lax.dot_general(
        hy, ne, (((1,), (1,)), ((), ())),
        preferred_element_type=jnp.float32)
    simt = jnp.maximum(_ALPHA * simt, 0.0)
    avg = jnp.mean(mask_ref[...], axis=0)
    simt = jnp.where(avg[None, :] < 0.5, _NEG, simt)
    m = jnp.max(simt, axis=0, keepdims=True)
    e = jnp.exp(simt - m)
    soft = e / jnp.sum(e, axis=0, keepdims=True)

    h = soft.shape[0]
    row = jax.lax.broadcasted_iota(jnp.int32, soft.shape, 0)
    v = soft
    vks, iks = [], []
    for k in range(_TOPK):
        mk = jnp.max(v, axis=0)
        ik = jnp.min(jnp.where(v == mk[None, :], row, h), axis=0)
        vks.append(mk)
        iks.append(ik)
        v = jnp.where(row == ik[None, :], -1.0, v)

    vf = jnp.concatenate([x.reshape(b // _L, _L) for x in vks], axis=1)
    vi = jnp.concatenate([x.astype(jnp.float32).reshape(b // _L, _L)
                          for x in iks], axis=1)
    for w in range(_TOPK):
        pw = p_ref[w]
        tf = jax.lax.dot_general(vf, pw, (((1,), (0,)), ((), ())),
                                 preferred_element_type=jnp.float32)
        ti = jax.lax.dot_general(vi, pw, (((1,), (0,)), ((), ())),
                                 preferred_element_type=jnp.float32)
        ew_ref[pl.Slice(w, b // _L, _TOPK), :] = tf
        ei_ref[pl.ds(1, 1), pl.Slice(w, b // _L, _TOPK), :] = (
            ti.astype(jnp.int32)[None])

    ploc = (jax.lax.broadcasted_iota(jnp.int32, (rows_out, _L), 0) * _L
            + jax.lax.broadcasted_iota(jnp.int32, (rows_out, _L), 1))
    ei_ref[pl.ds(0, 1), :, :] = (b * i + ((ploc * 21846) >> 16))[None]


def kernel(features, mask, node_embeds, hyper_embeds):
    seq_len = min(features.shape[1], node_embeds.shape[0])
    ne = node_embeds[:seq_len]
    dim = ne.shape[1]
    hnum = hyper_embeds.shape[0]
    nblk = seq_len // _BLOCK
    rpb = _TOPK * _BLOCK // _L
    nrows = nblk * rpb

    ew, ei = pl.pallas_call(
        _hyper_kernel,
        grid=(nblk,),
        in_specs=[
            pl.BlockSpec((mask.shape[0], _BLOCK), lambda i: (0, i)),
            pl.BlockSpec((_BLOCK, dim), lambda i: (i, 0)),
            pl.BlockSpec((hnum, dim), lambda i: (0, 0)),
            pl.BlockSpec((_TOPK, _TOPK * _L, _L), lambda i: (0, 0, 0)),
        ],
        out_specs=[
            pl.BlockSpec((rpb, _L), lambda i: (i, 0)),
            pl.BlockSpec((2, rpb, _L), lambda i: (0, i, 0)),
        ],
        out_shape=[
            jax.ShapeDtypeStruct((nrows, _L), jnp.float32),
            jax.ShapeDtypeStruct((2, nrows, _L), jnp.int32),
        ],
    )(mask, ne, hyper_embeds, _perm_mats())

    return (ei.reshape(2, -1), ew.reshape(-1))


# speedup vs baseline: 1.3695x; 1.0021x over previous
import numpy as np

import jax
import jax.numpy as jnp
from jax.experimental import pallas as pl
from jax.experimental.pallas import tpu as pltpu

_ALPHA = 1.0
_TOPK = 3
_BLOCK = 4096
_NEG = -1e9
_L = 128


def _perm_mats():
    p = np.zeros((_TOPK, _TOPK * _L, _L), np.float32)
    for w in range(_TOPK):
        for c in range(_L):
            q = _L * w + c
            p[w, _L * (q % _TOPK) + q // _TOPK, c] = 1.0
    return jnp.asarray(p)


def _hyper_kernel(mask_ref, ne_ref, hy_ref, p_ref, ew_ref, ei_ref):
    i = pl.program_id(0)
    b = ne_ref.shape[0]
    rows_out = ew_ref.shape[0]
    ne = ne_ref[...]
    hy = hy_ref[...]
    simt = jax.lax.dot_general(
        hy, ne, (((1,), (1,)), ((), ())),
        preferred_element_type=jnp.float32)
    simt = jnp.maximum(_ALPHA * simt, 0.0)
    avg = jnp.mean(mask_ref[...], axis=0)
    simt = jnp.where(avg[None, :] < 0.5, _NEG, simt)
    m = jnp.max(simt, axis=0, keepdims=True)
    e = jnp.exp(simt - m)
    soft = e / jnp.sum(e, axis=0, keepdims=True)

    h = soft.shape[0]
    row = jax.lax.broadcasted_iota(jnp.int32, soft.shape, 0)
    v = soft
    vks, iks = [], []
    for k in range(_TOPK):
        mk = jnp.max(v, axis=0)
        ik = jnp.min(jnp.where(v == mk[None, :], row, h), axis=0)
        vks.append(mk)
        iks.append(ik)
        v = jnp.where(row == ik[None, :], -1.0, v)

    vf = jnp.concatenate([x.reshape(b // _L, _L) for x in vks], axis=1)
    vi = jnp.concatenate([x.astype(jnp.float32).reshape(b // _L, _L)
                          for x in iks], axis=1)
    for w in range(_TOPK):
        pw = p_ref[w]
        tf = jax.lax.dot_general(vf, pw, (((1,), (0,)), ((), ())),
                                 preferred_element_type=jnp.float32)
        ti = jax.lax.dot_general(vi, pw, (((1,), (0,)), ((), ())),
                                 preferred_element_type=jnp.float32)
        ew_ref[pl.Slice(w, b // _L, _TOPK), :] = tf
        ei_ref[pl.ds(1, 1), pl.Slice(w, b // _L, _TOPK), :] = (
            ti.astype(jnp.int32)[None])

    ploc = (jax.lax.broadcasted_iota(jnp.int32, (rows_out, _L), 0) * _L
            + jax.lax.broadcasted_iota(jnp.int32, (rows_out, _L), 1))
    ei_ref[pl.ds(0, 1), :, :] = (b * i + ((ploc * 21846) >> 16))[None]


def kernel(features, mask, node_embeds, hyper_embeds):
    seq_len = min(features.shape[1], node_embeds.shape[0])
    ne = node_embeds[:seq_len]
    dim = ne.shape[1]
    hnum = hyper_embeds.shape[0]
    nblk = seq_len // _BLOCK
    rpb = _TOPK * _BLOCK // _L
    nrows = nblk * rpb

    ew, ei = pl.pallas_call(
        _hyper_kernel,
        grid=(nblk,),
        in_specs=[
            pl.BlockSpec((mask.shape[0], _BLOCK), lambda i: (0, i)),
            pl.BlockSpec((_BLOCK, dim), lambda i: (i, 0)),
            pl.BlockSpec((hnum, dim), lambda i: (0, 0)),
            pl.BlockSpec((_TOPK, _TOPK * _L, _L), lambda i: (0, 0, 0)),
        ],
        out_specs=[
            pl.BlockSpec((rpb, _L), lambda i: (i, 0)),
            pl.BlockSpec((2, rpb, _L), lambda i: (0, i, 0)),
        ],
        out_shape=[
            jax.ShapeDtypeStruct((nrows, _L), jnp.float32),
            jax.ShapeDtypeStruct((2, nrows, _L), jnp.int32),
        ],
    )(mask, ne, hyper_embeds, _perm_mats())

    return (ei.reshape(2, -1), ew.reshape(-1))
